# Initial kernel scaffold; baseline (speedup 1.0000x reference)
import jax
import jax.numpy as jnp
from jax.experimental import pallas as pl


def kernel(input_ids, token_type_ids, token_table, pos_table, type_table, gamma, beta):
    raise NotImplementedError("write your pallas kernel here")


# speedup vs baseline: 1.4591x; 1.4591x over previous
import functools

import jax
import jax.numpy as jnp
from jax import lax
from jax.experimental import pallas as pl
from jax.experimental.pallas import tpu as pltpu
from jax.experimental.pallas import tpu_sc as plsc

B, S, D = 4, 2048, 768
N = B * S
NW = 32
TPW = N // NW
CH = 64
TB = 256
SB = S // TB


def _sc_gather_body(ids_hbm, tok_hbm, out_hbm, idx_v, rows_v, sem):
    c = lax.axis_index("c")
    s = lax.axis_index("s")
    w = s * 2 + c
    for k in range(TPW // CH):
        base = w * TPW + k * CH
        pltpu.sync_copy(ids_hbm.at[pl.ds(base, CH)], idx_v)
        pltpu.async_copy(tok_hbm.at[idx_v], rows_v, sem).wait()
        pltpu.sync_copy(rows_v, out_hbm.at[pl.ds(base, CH), :])


def _sc_gather(ids, token_table):
    mesh = plsc.VectorSubcoreMesh(core_axis_name="c", subcore_axis_name="s")
    call = functools.partial(
        pl.kernel,
        mesh=mesh,
        out_type=jax.ShapeDtypeStruct((N, D), jnp.float32),
        scratch_types=[
            pltpu.VMEM((CH,), jnp.int32),
            pltpu.VMEM((CH, D), jnp.float32),
            pltpu.SemaphoreType.DMA,
        ],
    )(_sc_gather_body)
    return call(ids, token_table)


def _tc_ln_body(rows_ref, ttf_ref, pos_ref, type_ref, gamma_ref, beta_ref,
                out_ref):
    x = rows_ref[...] + pos_ref[...]
    ttf = ttf_ref[...]
    t0 = type_ref[0:1, :]
    t1 = type_ref[1:2, :]
    x = x + t0 + ttf * (t1 - t0)
    mean = jnp.mean(x, axis=1, keepdims=True)
    xc = x - mean
    var = jnp.mean(xc * xc, axis=1, keepdims=True)
    inv = lax.rsqrt(var + jnp.float32(1e-12))
    out_ref[...] = xc * inv * gamma_ref[...] + beta_ref[...]


def _tc_ln(rows, ttf, pos_table, type_table, gamma2, beta2):
    return pl.pallas_call(
        _tc_ln_body,
        grid=(N // TB,),
        in_specs=[
            pl.BlockSpec((TB, D), lambda i: (i, 0)),
            pl.BlockSpec((TB, 1), lambda i: (i, 0)),
            pl.BlockSpec((TB, D), lambda i: (i % SB, 0)),
            pl.BlockSpec((2, D), lambda i: (0, 0)),
            pl.BlockSpec((1, D), lambda i: (0, 0)),
            pl.BlockSpec((1, D), lambda i: (0, 0)),
        ],
        out_specs=pl.BlockSpec((TB, D), lambda i: (i, 0)),
        out_shape=jax.ShapeDtypeStruct((N, D), jnp.float32),
    )(rows, ttf, pos_table, type_table, gamma2, beta2)


def kernel(input_ids, token_type_ids, token_table, pos_table, type_table,
           gamma, beta):
    ids = input_ids.reshape(-1).astype(jnp.int32)
    ttf = token_type_ids.reshape(-1, 1).astype(jnp.float32)
    rows = _sc_gather(ids, token_table)
    out = _tc_ln(rows, ttf, pos_table, type_table,
                 gamma.reshape(1, D), beta.reshape(1, D))
    return out.reshape(B, S, D)


# speedup vs baseline: 1.5262x; 1.0460x over previous
import functools

import jax
import jax.numpy as jnp
from jax import lax
from jax.experimental import pallas as pl
from jax.experimental.pallas import tpu as pltpu
from jax.experimental.pallas import tpu_sc as plsc

B, S, D = 4, 2048, 768
N = B * S
NW = 32
TPW = N // NW
CH = 64
TB = 256
SB = S // TB


def _sc_gather_body(ids_hbm, tok_hbm, out_hbm, idx0, idx1, rows0, rows1,
                    sem0, sem1):
    c = lax.axis_index("c")
    s = lax.axis_index("s")
    w = s * 2 + c
    nk = TPW // CH
    idx = (idx0, idx1)
    rows = (rows0, rows1)
    sems = (sem0, sem1)
    copies = [None] * nk

    def start(k):
        base = w * TPW + k * CH
        pltpu.sync_copy(ids_hbm.at[pl.ds(base, CH)], idx[k % 2])
        copies[k] = pltpu.async_copy(tok_hbm.at[idx[k % 2]], rows[k % 2],
                                     sems[k % 2])

    start(0)
    start(1)
    for k in range(nk):
        base = w * TPW + k * CH
        copies[k].wait()
        pltpu.sync_copy(rows[k % 2], out_hbm.at[pl.ds(base, CH), :])
        if k + 2 < nk:
            start(k + 2)


def _sc_gather(ids, token_table):
    mesh = plsc.VectorSubcoreMesh(core_axis_name="c", subcore_axis_name="s")
    call = functools.partial(
        pl.kernel,
        mesh=mesh,
        out_type=jax.ShapeDtypeStruct((N, D), jnp.float32),
        scratch_types=[
            pltpu.VMEM((CH,), jnp.int32),
            pltpu.VMEM((CH,), jnp.int32),
            pltpu.VMEM((CH, D), jnp.float32),
            pltpu.VMEM((CH, D), jnp.float32),
            pltpu.SemaphoreType.DMA,
            pltpu.SemaphoreType.DMA,
        ],
    )(_sc_gather_body)
    return call(ids, token_table)


def _tc_ln_body(rows_ref, ttf_ref, pos_ref, type_ref, gamma_ref, beta_ref,
                out_ref):
    x = rows_ref[...] + pos_ref[...]
    ttf = ttf_ref[...]
    t0 = type_ref[0:1, :]
    t1 = type_ref[1:2, :]
    x = x + t0 + ttf * (t1 - t0)
    mean = jnp.mean(x, axis=1, keepdims=True)
    xc = x - mean
    var = jnp.mean(xc * xc, axis=1, keepdims=True)
    inv = lax.rsqrt(var + jnp.float32(1e-12))
    out_ref[...] = xc * inv * gamma_ref[...] + beta_ref[...]


def _tc_ln(rows, ttf, pos_table, type_table, gamma2, beta2):
    return pl.pallas_call(
        _tc_ln_body,
        grid=(SB, B),
        in_specs=[
            pl.BlockSpec((TB, D), lambda k, b: (b * SB + k, 0)),
            pl.BlockSpec((TB, 1), lambda k, b: (b * SB + k, 0)),
            pl.BlockSpec((TB, D), lambda k, b: (k, 0)),
            pl.BlockSpec((2, D), lambda k, b: (0, 0)),
            pl.BlockSpec((1, D), lambda k, b: (0, 0)),
            pl.BlockSpec((1, D), lambda k, b: (0, 0)),
        ],
        out_specs=pl.BlockSpec((TB, D), lambda k, b: (b * SB + k, 0)),
        out_shape=jax.ShapeDtypeStruct((N, D), jnp.float32),
    )(rows, ttf, pos_table, type_table, gamma2, beta2)


def kernel(input_ids, token_type_ids, token_table, pos_table, type_table,
           gamma, beta):
    ids = input_ids.reshape(-1).astype(jnp.int32)
    ttf = token_type_ids.reshape(-1, 1).astype(jnp.float32)
    rows = _sc_gather(ids, token_table)
    out = _tc_ln(rows, ttf, pos_table, type_table,
                 gamma.reshape(1, D), beta.reshape(1, D))
    return out.reshape(B, S, D)


# speedup vs baseline: 1.7796x; 1.1660x over previous
import functools

import jax
import jax.numpy as jnp
from jax import lax
from jax.experimental import pallas as pl
from jax.experimental.pallas import tpu as pltpu
from jax.experimental.pallas import tpu_sc as plsc

B, S, D = 4, 2048, 768
N = B * S
NW = 32
TPW = N // NW
CH = 64
TB = 512
SB = S // TB


def _sc_gather_body(ids_hbm, tok_hbm, out_hbm, idx0, idx1, rows0, rows1,
                    sem0, sem1):
    c = lax.axis_index("c")
    s = lax.axis_index("s")
    w = s * 2 + c
    nk = TPW // CH
    idx = (idx0, idx1)
    rows = (rows0, rows1)
    sems = (sem0, sem1)
    copies = [None] * nk

    def start(k):
        base = w * TPW + k * CH
        pltpu.sync_copy(ids_hbm.at[pl.ds(base, CH)], idx[k % 2])
        copies[k] = pltpu.async_copy(tok_hbm.at[idx[k % 2]], rows[k % 2],
                                     sems[k % 2])

    start(0)
    start(1)
    for k in range(nk):
        base = w * TPW + k * CH
        copies[k].wait()
        pltpu.sync_copy(rows[k % 2], out_hbm.at[pl.ds(base, CH), :])
        if k + 2 < nk:
            start(k + 2)


def _sc_gather(ids, token_table):
    mesh = plsc.VectorSubcoreMesh(core_axis_name="c", subcore_axis_name="s")
    call = functools.partial(
        pl.kernel,
        mesh=mesh,
        out_type=jax.ShapeDtypeStruct((N, D), jnp.float32),
        scratch_types=[
            pltpu.VMEM((CH,), jnp.int32),
            pltpu.VMEM((CH,), jnp.int32),
            pltpu.VMEM((CH, D), jnp.float32),
            pltpu.VMEM((CH, D), jnp.float32),
            pltpu.SemaphoreType.DMA,
            pltpu.SemaphoreType.DMA,
        ],
    )(_sc_gather_body)
    return call(ids, token_table)


def _tc_ln_body(rows_ref, ttf_ref, pos_ref, type_ref, gamma_ref, beta_ref,
                out_ref):
    x = rows_ref[...] + pos_ref[...]
    ttf = ttf_ref[...]
    t0 = type_ref[0:1, :]
    t1 = type_ref[1:2, :]
    x = x + t0 + ttf * (t1 - t0)
    mean = jnp.mean(x, axis=1, keepdims=True)
    xc = x - mean
    var = jnp.mean(xc * xc, axis=1, keepdims=True)
    inv = lax.rsqrt(var + jnp.float32(1e-12))
    out_ref[...] = xc * inv * gamma_ref[...] + beta_ref[...]


def _tc_ln(rows, ttf, pos_table, type_table, gamma2, beta2):
    return pl.pallas_call(
        _tc_ln_body,
        grid=(SB, B),
        in_specs=[
            pl.BlockSpec((TB, D), lambda k, b: (b * SB + k, 0)),
            pl.BlockSpec((TB, 1), lambda k, b: (b * SB + k, 0)),
            pl.BlockSpec((TB, D), lambda k, b: (k, 0)),
            pl.BlockSpec((2, D), lambda k, b: (0, 0)),
            pl.BlockSpec((1, D), lambda k, b: (0, 0)),
            pl.BlockSpec((1, D), lambda k, b: (0, 0)),
        ],
        out_specs=pl.BlockSpec((TB, D), lambda k, b: (b * SB + k, 0)),
        out_shape=jax.ShapeDtypeStruct((N, D), jnp.float32),
    )(rows, ttf, pos_table, type_table, gamma2, beta2)


def kernel(input_ids, token_type_ids, token_table, pos_table, type_table,
           gamma, beta):
    ids = input_ids.reshape(-1).astype(jnp.int32)
    ttf = token_type_ids.reshape(-1, 1).astype(jnp.float32)
    rows = _sc_gather(ids, token_table)
    out = _tc_ln(rows, ttf, pos_table, type_table,
                 gamma.reshape(1, D), beta.reshape(1, D))
    return out.reshape(B, S, D)


# speedup vs baseline: 1.8408x; 1.0344x over previous
import functools

import jax
import jax.numpy as jnp
from jax import lax
from jax.experimental import pallas as pl
from jax.experimental.pallas import tpu as pltpu
from jax.experimental.pallas import tpu_sc as plsc

B, S, D = 4, 2048, 768
N = B * S
NW = 32
TPW = N // NW
CH = 64
TB = 1024
SB = S // TB


def _sc_gather_body(ids_hbm, tok_hbm, out_hbm, idx0, idx1, rows0, rows1,
                    sem0, sem1):
    c = lax.axis_index("c")
    s = lax.axis_index("s")
    w = s * 2 + c
    nk = TPW // CH
    idx = (idx0, idx1)
    rows = (rows0, rows1)
    sems = (sem0, sem1)
    copies = [None] * nk

    def start(k):
        base = w * TPW + k * CH
        pltpu.sync_copy(ids_hbm.at[pl.ds(base, CH)], idx[k % 2])
        copies[k] = pltpu.async_copy(tok_hbm.at[idx[k % 2]], rows[k % 2],
                                     sems[k % 2])

    start(0)
    start(1)
    for k in range(nk):
        base = w * TPW + k * CH
        copies[k].wait()
        pltpu.sync_copy(rows[k % 2], out_hbm.at[pl.ds(base, CH), :])
        if k + 2 < nk:
            start(k + 2)


def _sc_gather(ids, token_table):
    mesh = plsc.VectorSubcoreMesh(core_axis_name="c", subcore_axis_name="s")
    call = functools.partial(
        pl.kernel,
        mesh=mesh,
        out_type=jax.ShapeDtypeStruct((N, D), jnp.float32),
        scratch_types=[
            pltpu.VMEM((CH,), jnp.int32),
            pltpu.VMEM((CH,), jnp.int32),
            pltpu.VMEM((CH, D), jnp.float32),
            pltpu.VMEM((CH, D), jnp.float32),
            pltpu.SemaphoreType.DMA,
            pltpu.SemaphoreType.DMA,
        ],
    )(_sc_gather_body)
    return call(ids, token_table)


def _tc_ln_body(rows_ref, ttf_ref, pos_ref, type_ref, gamma_ref, beta_ref,
                out_ref):
    x = rows_ref[...] + pos_ref[...]
    ttf = ttf_ref[...]
    t0 = type_ref[0:1, :]
    t1 = type_ref[1:2, :]
    x = x + t0 + ttf * (t1 - t0)
    mean = jnp.mean(x, axis=1, keepdims=True)
    xc = x - mean
    var = jnp.mean(xc * xc, axis=1, keepdims=True)
    inv = lax.rsqrt(var + jnp.float32(1e-12))
    out_ref[...] = xc * inv * gamma_ref[...] + beta_ref[...]


def _tc_ln(rows, ttf, pos_table, type_table, gamma2, beta2):
    return pl.pallas_call(
        _tc_ln_body,
        grid=(SB, B),
        in_specs=[
            pl.BlockSpec((TB, D), lambda k, b: (b * SB + k, 0)),
            pl.BlockSpec((TB, 1), lambda k, b: (b * SB + k, 0)),
            pl.BlockSpec((TB, D), lambda k, b: (k, 0)),
            pl.BlockSpec((2, D), lambda k, b: (0, 0)),
            pl.BlockSpec((1, D), lambda k, b: (0, 0)),
            pl.BlockSpec((1, D), lambda k, b: (0, 0)),
        ],
        out_specs=pl.BlockSpec((TB, D), lambda k, b: (b * SB + k, 0)),
        out_shape=jax.ShapeDtypeStruct((N, D), jnp.float32),
    )(rows, ttf, pos_table, type_table, gamma2, beta2)


def kernel(input_ids, token_type_ids, token_table, pos_table, type_table,
           gamma, beta):
    ids = input_ids.reshape(-1).astype(jnp.int32)
    ttf = token_type_ids.reshape(-1, 1).astype(jnp.float32)
    rows = _sc_gather(ids, token_table)
    out = _tc_ln(rows, ttf, pos_table, type_table,
                 gamma.reshape(1, D), beta.reshape(1, D))
    return out.reshape(B, S, D)
